# baseline (speedup 1.0000x reference)
import functools

import jax
import jax.numpy as jnp
from jax import lax
from jax.experimental import pallas as pl
from jax.experimental.pallas import tpu as pltpu
from jax.experimental.pallas import tpu_sc as plsc

N = 4096
L = 16
NC = 2
NS = 16
NW = NC * NS
ROWS_PER_TILE = N // NW
GROUPS = ROWS_PER_TILE // L
INF = float("inf")
WB_OFF = 8
CH = 8


def _nn_pool_body(x2_h, y2_h, x1_h, y1_h, wb_h, out_h,
                  x2, y2, vx, vy, wb, out_v):
    cid = lax.axis_index("c")
    sid = lax.axis_index("s")
    wid = sid * NC + cid
    row_base = wid * ROWS_PER_TILE

    pltpu.sync_copy(x2_h, x2)
    pltpu.sync_copy(y2_h, y2)
    pltpu.sync_copy(x1_h, vx)
    pltpu.sync_copy(y1_h, vy)
    pltpu.sync_copy(wb_h, wb)

    lanes = lax.broadcasted_iota(jnp.int32, (L,), 0)

    def vel_body(k, _):
        s = pl.ds(k * L, L)
        vx[s] = x2[s] - vx[s]
        vy[s] = y2[s] - vy[s]
        return 0

    lax.fori_loop(0, N // L, vel_body, 0)

    def splat(ref, idx):
        return plsc.load_gather(ref, [jnp.full((L,), idx, jnp.int32)])

    def insert_one(carry, d, jv):
        v1, v2, v3, v4, a1, a2, a3, a4 = carry
        c1, c2, c3, c4 = d < v1, d < v2, d < v3, d < v4
        nv4 = jnp.where(c4, jnp.where(c3, v3, d), v4)
        na4 = jnp.where(c4, jnp.where(c3, a3, jv), a4)
        nv3 = jnp.where(c3, jnp.where(c2, v2, d), v3)
        na3 = jnp.where(c3, jnp.where(c2, a2, jv), a3)
        nv2 = jnp.where(c2, jnp.where(c1, v1, d), v2)
        na2 = jnp.where(c2, jnp.where(c1, a1, jv), a2)
        nv1 = jnp.where(c1, d, v1)
        na1 = jnp.where(c1, jv, a1)
        return nv1, nv2, nv3, nv4, na1, na2, na3, na4

    def do_group(g, _):
        i0 = row_base + g * L
        ivec = i0 + lanes
        xi = x2[pl.ds(i0, L)]
        yi = y2[pl.ds(i0, L)]

        def chunk_dists(j0):
            xc = x2[pl.ds(j0, L)]
            yc = y2[pl.ds(j0, L)]
            out = []
            for k in range(L):
                jv = jnp.full((L,), j0 + k, jnp.int32)
                dx = xc[k] - xi
                dy = yc[k] - yi
                out.append((dx * dx + dy * dy, jv))
            return out

        def chunk_body(jc, carry):
            for d, jv in chunk_dists(jc * L):
                carry = insert_one(carry, d, jv)
            return carry

        def self_chunk(carry):
            for d, jv in chunk_dists(i0):
                d = jnp.where(jv == ivec, INF, d)
                carry = insert_one(carry, d, jv)
            return carry

        inf_v = jnp.full((L,), INF)
        zer = jnp.zeros((L,), jnp.int32)
        carry = (inf_v, inf_v, inf_v, inf_v, zer, zer, zer, zer)
        carry = plsc.parallel_loop(0, i0 // L, 1, unroll=4, carry=carry)(chunk_body)
        carry = self_chunk(carry)
        carry = plsc.parallel_loop(i0 // L + 1, N // L, 1, unroll=4, carry=carry)(chunk_body)
        _, _, _, _, a1, a2, a3, a4 = carry

        vxi = vx[pl.ds(i0, L)]
        vyi = vy[pl.ds(i0, L)]
        rvec = g * L + lanes
        feats = []
        for an in (a1, a2, a3, a4):
            feats.append((plsc.load_gather(x2, [an]) - xi,
                          plsc.load_gather(y2, [an]) - yi,
                          plsc.load_gather(vx, [an]) - vxi,
                          plsc.load_gather(vy, [an]) - vyi))
        for o in range(8):
            w0 = splat(wb, WB_OFF + 0 * 8 + o)
            w1 = splat(wb, WB_OFF + 1 * 8 + o)
            w2 = splat(wb, WB_OFF + 2 * 8 + o)
            w3 = splat(wb, WB_OFF + 3 * 8 + o)
            bo = splat(wb, WB_OFF + 32 + o)
            for n, (fx, fy, fvx, fvy) in enumerate(feats):
                acc = fx * w0 + fy * w1 + fvx * w2 + fvy * w3 + bo
                acc = jnp.maximum(acc, 0.0)
                plsc.store_scatter(out_v, [rvec, jnp.full((L,), n * 8 + o, jnp.int32)], acc)
        return 0

    lax.fori_loop(0, GROUPS, do_group, 0)
    pltpu.sync_copy(out_v, out_h.at[pl.ds(row_base, ROWS_PER_TILE)])


@jax.jit
def _nn_pool(x2, y2, x1, y1, wb):
    mesh = plsc.VectorSubcoreMesh(core_axis_name="c", subcore_axis_name="s",
                                  num_cores=NC)
    f = functools.partial(
        pl.kernel,
        mesh=mesh,
        compiler_params=pltpu.CompilerParams(needs_layout_passes=False),
        out_type=jax.ShapeDtypeStruct((N, 32), jnp.float32),
        scratch_types=[
            pltpu.VMEM((N,), jnp.float32),
            pltpu.VMEM((N,), jnp.float32),
            pltpu.VMEM((N,), jnp.float32),
            pltpu.VMEM((N,), jnp.float32),
            pltpu.VMEM((48,), jnp.float32),
            pltpu.VMEM((ROWS_PER_TILE, 32), jnp.float32),
        ],
    )(_nn_pool_body)
    return f(x2, y2, x1, y1, wb)


def kernel(_, obs1, obs2, W, b):
    x2 = obs2[:, 0]
    y2 = obs2[:, 1]
    x1 = obs1[:, 0]
    y1 = obs1[:, 1]
    wb = jnp.concatenate([jnp.zeros((WB_OFF,), jnp.float32), W.reshape(32), b])
    return _nn_pool(x2, y2, x1, y1, wb)


# speedup vs baseline: 1.0048x; 1.0048x over previous
import functools

import jax
import jax.numpy as jnp
from jax import lax
from jax.experimental import pallas as pl
from jax.experimental.pallas import tpu as pltpu
from jax.experimental.pallas import tpu_sc as plsc

N = 4096
L = 16
NC = 2
NS = 16
NW = NC * NS
ROWS_PER_TILE = N // NW
GROUPS = ROWS_PER_TILE // L
INF = float("inf")
WB_OFF = 8
CH = 8


def _nn_pool_body(x2_h, y2_h, x1_h, y1_h, wb_h, out_h,
                  x2, y2, vx, vy, wb, out_v):
    cid = lax.axis_index("c")
    sid = lax.axis_index("s")
    wid = sid * NC + cid
    row_base = wid * ROWS_PER_TILE

    pltpu.sync_copy(x2_h, x2)
    pltpu.sync_copy(y2_h, y2)
    pltpu.sync_copy(x1_h, vx)
    pltpu.sync_copy(y1_h, vy)
    pltpu.sync_copy(wb_h, wb)

    lanes = lax.broadcasted_iota(jnp.int32, (L,), 0)

    def vel_body(k, _):
        s = pl.ds(k * L, L)
        vx[s] = x2[s] - vx[s]
        vy[s] = y2[s] - vy[s]
        return 0

    lax.fori_loop(0, N // L, vel_body, 0)

    def splat(ref, idx):
        return plsc.load_gather(ref, [jnp.full((L,), idx, jnp.int32)])

    def insert_one(carry, d, jv):
        v1, v2, v3, v4, a1, a2, a3, a4 = carry
        c1, c2, c3, c4 = d < v1, d < v2, d < v3, d < v4
        nv4 = jnp.where(c4, jnp.where(c3, v3, d), v4)
        na4 = jnp.where(c4, jnp.where(c3, a3, jv), a4)
        nv3 = jnp.where(c3, jnp.where(c2, v2, d), v3)
        na3 = jnp.where(c3, jnp.where(c2, a2, jv), a3)
        nv2 = jnp.where(c2, jnp.where(c1, v1, d), v2)
        na2 = jnp.where(c2, jnp.where(c1, a1, jv), a2)
        nv1 = jnp.where(c1, d, v1)
        na1 = jnp.where(c1, jv, a1)
        return nv1, nv2, nv3, nv4, na1, na2, na3, na4

    def do_group(g, _):
        i0 = row_base + g * L
        ivec = i0 + lanes
        xi = x2[pl.ds(i0, L)]
        yi = y2[pl.ds(i0, L)]

        def chunk_dists(j0):
            xc = x2[pl.ds(j0, L)]
            yc = y2[pl.ds(j0, L)]
            out = []
            for k in range(L):
                jv = jnp.full((L,), j0 + k, jnp.int32)
                dx = xc[k] - xi
                dy = yc[k] - yi
                out.append((dx * dx + dy * dy, jv))
            return out

        def chunk_body(jc, carry):
            for d, jv in chunk_dists(jc * L):
                carry = insert_one(carry, d, jv)
            return carry

        def self_chunk(carry):
            for d, jv in chunk_dists(i0):
                d = jnp.where(jv == ivec, INF, d)
                carry = insert_one(carry, d, jv)
            return carry

        inf_v = jnp.full((L,), INF)
        zer = jnp.zeros((L,), jnp.int32)
        carry = (inf_v, inf_v, inf_v, inf_v, zer, zer, zer, zer)
        carry = plsc.parallel_loop(0, i0 // L, 1, unroll=2, carry=carry)(chunk_body)
        carry = self_chunk(carry)
        carry = plsc.parallel_loop(i0 // L + 1, N // L, 1, unroll=2, carry=carry)(chunk_body)
        _, _, _, _, a1, a2, a3, a4 = carry

        vxi = vx[pl.ds(i0, L)]
        vyi = vy[pl.ds(i0, L)]
        rvec = g * L + lanes
        feats = []
        for an in (a1, a2, a3, a4):
            feats.append((plsc.load_gather(x2, [an]) - xi,
                          plsc.load_gather(y2, [an]) - yi,
                          plsc.load_gather(vx, [an]) - vxi,
                          plsc.load_gather(vy, [an]) - vyi))
        for o in range(8):
            w0 = splat(wb, WB_OFF + 0 * 8 + o)
            w1 = splat(wb, WB_OFF + 1 * 8 + o)
            w2 = splat(wb, WB_OFF + 2 * 8 + o)
            w3 = splat(wb, WB_OFF + 3 * 8 + o)
            bo = splat(wb, WB_OFF + 32 + o)
            for n, (fx, fy, fvx, fvy) in enumerate(feats):
                acc = fx * w0 + fy * w1 + fvx * w2 + fvy * w3 + bo
                acc = jnp.maximum(acc, 0.0)
                plsc.store_scatter(out_v, [rvec, jnp.full((L,), n * 8 + o, jnp.int32)], acc)
        return 0

    lax.fori_loop(0, GROUPS, do_group, 0)
    pltpu.sync_copy(out_v, out_h.at[pl.ds(row_base, ROWS_PER_TILE)])


@jax.jit
def _nn_pool(x2, y2, x1, y1, wb):
    mesh = plsc.VectorSubcoreMesh(core_axis_name="c", subcore_axis_name="s",
                                  num_cores=NC)
    f = functools.partial(
        pl.kernel,
        mesh=mesh,
        compiler_params=pltpu.CompilerParams(needs_layout_passes=False),
        out_type=jax.ShapeDtypeStruct((N, 32), jnp.float32),
        scratch_types=[
            pltpu.VMEM((N,), jnp.float32),
            pltpu.VMEM((N,), jnp.float32),
            pltpu.VMEM((N,), jnp.float32),
            pltpu.VMEM((N,), jnp.float32),
            pltpu.VMEM((48,), jnp.float32),
            pltpu.VMEM((ROWS_PER_TILE, 32), jnp.float32),
        ],
    )(_nn_pool_body)
    return f(x2, y2, x1, y1, wb)


def kernel(_, obs1, obs2, W, b):
    x2 = obs2[:, 0]
    y2 = obs2[:, 1]
    x1 = obs1[:, 0]
    y1 = obs1[:, 1]
    wb = jnp.concatenate([jnp.zeros((WB_OFF,), jnp.float32), W.reshape(32), b])
    return _nn_pool(x2, y2, x1, y1, wb)
